# Initial kernel scaffold; baseline (speedup 1.0000x reference)
import jax
import jax.numpy as jnp
from jax.experimental import pallas as pl


def kernel(loc_data, conf_data, priors, targets):
    raise NotImplementedError("write your pallas kernel here")


# speedup vs baseline: 26.8113x; 26.8113x over previous
import jax
import jax.numpy as jnp
from jax import lax
from jax.experimental import pallas as pl
from jax.experimental.pallas import tpu as pltpu

_B, _P, _C, _O = 32, 24564, 21, 32
_PADP = 24576
_G, _L = 192, 128
_THRESH = 0.5
_NEGPOS = 3
_VAR0, _VAR1 = 0.1, 0.2
_BIG = 2**30


def _image_kernel(tgt_ref, loc_ref, conf_ref, pri_ref, out_ref):
    f32 = jnp.float32
    lin = (lax.broadcasted_iota(jnp.int32, (_G, _L), 0) * _L
           + lax.broadcasted_iota(jnp.int32, (_G, _L), 1))
    valid = lin < _P

    pcx = pri_ref[0]
    pcy = pri_ref[1]
    pw = pri_ref[2]
    ph = pri_ref[3]
    px1 = pcx - pw * 0.5
    py1 = pcy - ph * 0.5
    px2 = pcx + pw * 0.5
    py2 = pcy + ph * 0.5
    parea = pw * ph

    t_x1 = [tgt_ref[0, t, 0] for t in range(_O)]
    t_y1 = [tgt_ref[0, t, 1] for t in range(_O)]
    t_x2 = [tgt_ref[0, t, 2] for t in range(_O)]
    t_y2 = [tgt_ref[0, t, 3] for t in range(_O)]
    t_lab = [tgt_ref[0, t, 4] for t in range(_O)]

    bt_ovl = jnp.full((_G, _L), -2.0, f32)
    bt_idx = jnp.zeros((_G, _L), jnp.int32)
    bpi = []
    for t in range(_O):
        iw = jnp.maximum(jnp.minimum(t_x2[t], px2) - jnp.maximum(t_x1[t], px1), 0.0)
        ih = jnp.maximum(jnp.minimum(t_y2[t], py2) - jnp.maximum(t_y1[t], py1), 0.0)
        inter = iw * ih
        tarea = (t_x2[t] - t_x1[t]) * (t_y2[t] - t_y1[t])
        ovl = inter / (tarea + parea - inter)
        ovl = jnp.where(valid, ovl, -1.0)
        take = ovl > bt_ovl
        bt_idx = jnp.where(take, t, bt_idx)
        bt_ovl = jnp.where(take, ovl, bt_ovl)
        mx = jnp.max(ovl)
        bpi.append(jnp.min(jnp.where(ovl == mx, lin, _BIG)))

    for t in range(_O):
        m = lin == bpi[t]
        bt_idx = jnp.where(m, t, bt_idx)
        bt_ovl = jnp.where(m, 2.0, bt_ovl)

    mx1 = jnp.zeros((_G, _L), f32)
    my1 = jnp.zeros((_G, _L), f32)
    mx2 = jnp.zeros((_G, _L), f32)
    my2 = jnp.zeros((_G, _L), f32)
    mlab = jnp.zeros((_G, _L), f32)
    for t in range(_O):
        s = bt_idx == t
        mx1 = jnp.where(s, t_x1[t], mx1)
        my1 = jnp.where(s, t_y1[t], my1)
        mx2 = jnp.where(s, t_x2[t], mx2)
        my2 = jnp.where(s, t_y2[t], my2)
        mlab = jnp.where(s, t_lab[t], mlab)

    pos = jnp.logical_and(bt_ovl >= _THRESH, mlab > 0.0)
    pos = jnp.logical_and(pos, valid)

    g_cx = ((mx1 + mx2) * 0.5 - pcx) / (_VAR0 * pw)
    g_cy = ((my1 + my2) * 0.5 - pcy) / (_VAR0 * ph)
    safe_w = jnp.where(pos, (mx2 - mx1) / pw, 1.0)
    safe_h = jnp.where(pos, (my2 - my1) / ph, 1.0)
    g_w = jnp.log(safe_w) / _VAR1
    g_h = jnp.log(safe_h) / _VAR1

    def sl1(d):
        a = jnp.abs(d)
        return jnp.where(a < 1.0, 0.5 * d * d, a - 0.5)

    l_elem = (sl1(loc_ref[0, 0] - g_cx) + sl1(loc_ref[0, 1] - g_cy)
              + sl1(loc_ref[0, 2] - g_w) + sl1(loc_ref[0, 3] - g_h))
    loss_l = jnp.sum(jnp.where(pos, l_elem, 0.0))

    rowmax = conf_ref[0, 0]
    for c in range(1, _C):
        rowmax = jnp.maximum(rowmax, conf_ref[0, c])
    conf_t = jnp.where(bt_ovl < _THRESH, 0, mlab.astype(jnp.int32))
    sexp = jnp.zeros((_G, _L), f32)
    gath = jnp.zeros((_G, _L), f32)
    for c in range(_C):
        x = conf_ref[0, c]
        sexp = sexp + jnp.exp(x - rowmax)
        gath = jnp.where(conf_t == c, x, gath)
    ce = jnp.log(sexp) + rowmax - gath
    ce = jnp.where(valid, ce, 0.0)

    pos_ce = jnp.sum(jnp.where(pos, ce, 0.0))
    npos = jnp.sum(pos.astype(jnp.int32))
    k = jnp.minimum(_NEGPOS * npos, _P - 1)

    loss_c = jnp.maximum(jnp.where(jnp.logical_or(pos, jnp.logical_not(valid)),
                                   0.0, ce), 0.0)
    bits = lax.bitcast_convert_type(loss_c, jnp.int32)

    def bis(i, tacc):
        cand = tacc | lax.shift_left(jnp.int32(1), 30 - i)
        cnt = jnp.sum(jnp.where(bits >= cand, 1, 0))
        return jnp.where(cnt >= k, cand, tacc)

    tbits = lax.fori_loop(0, 31, bis, jnp.int32(0))
    above = bits > tbits
    m_cnt = jnp.sum(jnp.where(above, 1, 0))
    sum_gt = jnp.sum(jnp.where(above, loss_c, 0.0))
    tval = lax.bitcast_convert_type(tbits, f32)
    r = k - m_cnt
    neg_c = sum_gt + jnp.where(r > 0, r.astype(f32) * tval, 0.0)

    lane = lax.broadcasted_iota(jnp.int32, (1, _L), 1)
    row = jnp.where(lane == 0, loss_l,
                    jnp.where(lane == 1, pos_ce,
                              jnp.where(lane == 2, neg_c,
                                        jnp.where(lane == 3, npos.astype(f32),
                                                  0.0))))
    out_ref[0] = row


def kernel(loc_data, conf_data, priors, targets):
    pad = _PADP - _P
    dummy = jnp.tile(jnp.array([[-10.0, -10.0, 0.1, 0.1]], jnp.float32), (pad, 1))
    pri = jnp.concatenate([priors, dummy], axis=0).T.reshape(4, _G, _L)
    loc = jnp.pad(loc_data, ((0, 0), (0, pad), (0, 0))).transpose(0, 2, 1)
    loc = loc.reshape(_B, 4, _G, _L)
    conf = jnp.pad(conf_data, ((0, 0), (0, pad), (0, 0))).transpose(0, 2, 1)
    conf = conf.reshape(_B, _C, _G, _L)

    partial = pl.pallas_call(
        _image_kernel,
        grid=(_B,),
        in_specs=[
            pl.BlockSpec((1, _O, 5), lambda b: (b, 0, 0),
                         memory_space=pltpu.SMEM),
            pl.BlockSpec((1, 4, _G, _L), lambda b: (b, 0, 0, 0)),
            pl.BlockSpec((1, _C, _G, _L), lambda b: (b, 0, 0, 0)),
            pl.BlockSpec((4, _G, _L), lambda b: (0, 0, 0)),
        ],
        out_specs=pl.BlockSpec((1, 1, _L), lambda b: (b, 0, 0)),
        out_shape=jax.ShapeDtypeStruct((_B, 1, _L), jnp.float32),
    )(targets, loc, conf, pri)

    part = partial.reshape(_B, _L)
    loss_l = jnp.sum(part[:, 0])
    loss_c = jnp.sum(part[:, 1]) + jnp.sum(part[:, 2])
    n = jnp.maximum(jnp.sum(part[:, 3]), 1.0)
    return (loss_l / n, loss_c / n)
